# Initial kernel scaffold; baseline (speedup 1.0000x reference)
import jax
import jax.numpy as jnp
from jax.experimental import pallas as pl


def kernel(x, y, F_active, conv_w):
    raise NotImplementedError("write your pallas kernel here")


# speedup vs baseline: 1.9471x; 1.9471x over previous
import functools

import jax
import jax.numpy as jnp
from jax import lax
from jax.experimental import pallas as pl
from jax.experimental.pallas import tpu as pltpu
from jax.experimental.pallas import tpu_sc as plsc

N_CELLS = 2
C = 128
H = 256
W = 256
NPTS = 131072

NC = 2
NS = 16
NW = NC * NS
PTS_PER_W = NPTS // NW
CHUNK = 64
N_CHUNKS = PTS_PER_W // CHUNK
GROUPS = CHUNK // 16
ROWS_PER_GROUP = 16 * 8

_CONV_T = 8


def _conv_tanh_body(wt_ref, prev_ref, cur_ref, nxt_ref, out_ref):
    i = pl.program_id(1)
    ni = pl.num_programs(1)
    t = _CONV_T
    prev = prev_ref[0]
    cur = cur_ref[0]
    nxt = nxt_ref[0]
    top = jnp.where(i > 0, prev[t - 1:t], 0.0)
    bot = jnp.where(i < ni - 1, nxt[0:1], 0.0)
    ext = jnp.concatenate([top, cur, bot], axis=0)
    acc = jnp.zeros((t, W, C), jnp.float32)
    zcol = jnp.zeros((t, 1, C), jnp.float32)
    for dh in range(3):
        sl = ext[dh:dh + t]
        for dw in range(3):
            if dw == 0:
                sh = jnp.concatenate([zcol, sl[:, :W - 1, :]], axis=1)
            elif dw == 2:
                sh = jnp.concatenate([sl[:, 1:, :], zcol], axis=1)
            else:
                sh = sl
            acc = acc + sh * wt_ref[dh * 3 + dw][None, None, :]
    out_ref[0] = jnp.tanh(acc)


def _conv_tanh(f_nhwc, wt9):
    nblk = H // _CONV_T
    blk = (1, _CONV_T, W, C)
    return pl.pallas_call(
        _conv_tanh_body,
        grid=(N_CELLS, nblk),
        in_specs=[
            pl.BlockSpec((9, C), lambda n, i: (0, 0)),
            pl.BlockSpec(blk, lambda n, i: (n, jnp.maximum(i - 1, 0), 0, 0)),
            pl.BlockSpec(blk, lambda n, i: (n, i, 0, 0)),
            pl.BlockSpec(blk, lambda n, i: (n, jnp.minimum(i + 1, nblk - 1), 0, 0)),
        ],
        out_specs=pl.BlockSpec(blk, lambda n, i: (n, i, 0, 0)),
        out_shape=jax.ShapeDtypeStruct((N_CELLS, H, W, C), jnp.float32),
    )(wt9, f_nhwc, f_nhwc, f_nhwc)


def _idxwts_body(x_ref, y_ref, idx_ref, wts_ref):
    xv = x_ref[...]
    yv = y_ref[...]
    ix0 = xv * jnp.float32(W - 2) * 0.5
    iy0 = yv * jnp.float32(H - 2)
    for n in range(N_CELLS):
        off = jnp.float32(n / N_CELLS)
        ix = ix0 + off
        iy = iy0 + off
        ixl = jnp.floor(ix)
        iyt = jnp.floor(iy)
        fx = ix - ixl
        fy = iy - iyt
        wxr = 0.5 - 0.5 * jnp.cos(jnp.pi * fx)
        wxl = 1.0 - wxr
        wyb = 0.5 - 0.5 * jnp.cos(jnp.pi * fy)
        wyt = 1.0 - wyb
        ixl_i = jnp.clip(ixl, 0, W - 1).astype(jnp.int32)
        ixr_i = jnp.clip(ixl + 1.0, 0, W - 1).astype(jnp.int32)
        iyt_i = jnp.clip(iyt, 0, H - 1).astype(jnp.int32)
        iyb_i = jnp.clip(iyt + 1.0, 0, H - 1).astype(jnp.int32)
        base = n * H * W
        idx_ref[4 * n + 0] = base + iyt_i * W + ixl_i
        idx_ref[4 * n + 1] = base + iyt_i * W + ixr_i
        idx_ref[4 * n + 2] = base + iyb_i * W + ixl_i
        idx_ref[4 * n + 3] = base + iyb_i * W + ixr_i
        wts_ref[4 * n + 0] = wxl * wyt
        wts_ref[4 * n + 1] = wxr * wyt
        wts_ref[4 * n + 2] = wxl * wyb
        wts_ref[4 * n + 3] = wxr * wyb


def _idxwts(xb, yb):
    rows = NPTS // 128
    rblk = 128
    nblk = rows // rblk
    return pl.pallas_call(
        _idxwts_body,
        grid=(nblk,),
        in_specs=[
            pl.BlockSpec((rblk, 128), lambda i: (i, 0)),
            pl.BlockSpec((rblk, 128), lambda i: (i, 0)),
        ],
        out_specs=[
            pl.BlockSpec((8, rblk, 128), lambda i: (0, i, 0)),
            pl.BlockSpec((8, rblk, 128), lambda i: (0, i, 0)),
        ],
        out_shape=[
            jax.ShapeDtypeStruct((8, rows, 128), jnp.int32),
            jax.ShapeDtypeStruct((8, rows, 128), jnp.float32),
        ],
    )(xb, yb)


def _sc_interp_body(table_hbm, idx_hbm, wts_hbm, out_hbm,
                    idx_v, wts_v, rows_v, out_v, sem):
    wid = lax.axis_index("s") * NC + lax.axis_index("c")
    iota = lax.iota(jnp.int32, 16)

    def chunk_body(k, carry):
        g4 = wid * (N_CHUNKS * GROUPS) + k * GROUPS
        pltpu.sync_copy(idx_hbm.at[pl.ds(g4, GROUPS)], idx_v)
        pltpu.sync_copy(wts_hbm.at[pl.ds(g4 * 8, GROUPS * 8)], wts_v)
        cps = [pltpu.async_copy(
                   table_hbm.at[idx_v.at[i]],
                   rows_v.at[pl.ds(i * ROWS_PER_GROUP, ROWS_PER_GROUP)], sem)
               for i in range(GROUPS)]
        for cp in cps:
            cp.wait()
        for pc in range(GROUPS):
            wvecs = [wts_v[pc * 8 + j] for j in range(8)]
            rowb = iota * 8 + pc * ROWS_PER_GROUP

            def cbody(c, _, wvecs=wvecs, rowb=rowb, pc=pc):
                col = jnp.broadcast_to(c, (16,))
                acc = jnp.zeros((16,), jnp.float32)
                for j in range(8):
                    g = plsc.load_gather(rows_v, [rowb + j, col])
                    acc = acc + wvecs[j] * g
                plsc.store_scatter(out_v, [iota + pc * 16, col], acc)
                return 0

            lax.fori_loop(0, C, cbody, 0)
        base_pt = wid * PTS_PER_W + k * CHUNK
        pltpu.sync_copy(out_v, out_hbm.at[pl.ds(base_pt, CHUNK)])
        return carry

    lax.fori_loop(0, N_CHUNKS, chunk_body, 0)


@functools.lru_cache(maxsize=1)
def _sc_interp():
    return pl.kernel(
        _sc_interp_body,
        out_type=jax.ShapeDtypeStruct((NPTS, C), jnp.float32),
        mesh=plsc.VectorSubcoreMesh(
            core_axis_name="c", subcore_axis_name="s",
            num_cores=NC, num_subcores=NS),
        compiler_params=pltpu.CompilerParams(needs_layout_passes=False),
        scratch_types=[
            pltpu.VMEM((GROUPS, 128), jnp.int32),
            pltpu.VMEM((GROUPS * 8, 16), jnp.float32),
            pltpu.VMEM((GROUPS * ROWS_PER_GROUP, C), jnp.float32),
            pltpu.VMEM((CHUNK, C), jnp.float32),
            pltpu.SemaphoreType.DMA,
        ],
    )


def kernel(x, y, F_active, conv_w):
    f_nhwc = jnp.transpose(F_active, (0, 2, 3, 1))
    wt9 = conv_w.reshape(C, 9).T
    table = _conv_tanh(f_nhwc, wt9).reshape(N_CELLS * H * W, C)

    xb = x.reshape(NPTS // 128, 128)
    yb = y.reshape(NPTS // 128, 128)
    idx_out, wts_out = _idxwts(xb, yb)

    idx_pj = idx_out.reshape(8, NPTS).T
    idxs = idx_pj.reshape(NPTS // 16, 16 * 8)
    wts_pj = wts_out.reshape(8, NPTS).T
    wtsr = (wts_pj.reshape(NPTS // 16, 16, 8)
            .transpose(0, 2, 1)
            .reshape(NPTS // 16 * 8, 16))

    return _sc_interp()(table, idxs, wtsr)


# speedup vs baseline: 2.6736x; 1.3731x over previous
import functools

import jax
import jax.numpy as jnp
from jax import lax
from jax.experimental import pallas as pl
from jax.experimental.pallas import tpu as pltpu
from jax.experimental.pallas import tpu_sc as plsc

N_CELLS = 2
C = 128
H = 256
W = 256
NPTS = 131072

NC = 2
NS = 16
NW = NC * NS
PTS_PER_W = NPTS // NW
CHUNK = 32
N_CHUNKS = PTS_PER_W // CHUNK
GROUPS = CHUNK // 16
ROWS_PER_GROUP = 16 * 8

_CONV_T = 8


def _conv_tanh_body(wt_ref, prev_ref, cur_ref, nxt_ref, out_ref):
    i = pl.program_id(1)
    ni = pl.num_programs(1)
    t = _CONV_T
    prev = prev_ref[0]
    cur = cur_ref[0]
    nxt = nxt_ref[0]
    top = jnp.where(i > 0, prev[t - 1:t], 0.0)
    bot = jnp.where(i < ni - 1, nxt[0:1], 0.0)
    ext = jnp.concatenate([top, cur, bot], axis=0)
    acc = jnp.zeros((t, W, C), jnp.float32)
    zcol = jnp.zeros((t, 1, C), jnp.float32)
    for dh in range(3):
        sl = ext[dh:dh + t]
        for dw in range(3):
            if dw == 0:
                sh = jnp.concatenate([zcol, sl[:, :W - 1, :]], axis=1)
            elif dw == 2:
                sh = jnp.concatenate([sl[:, 1:, :], zcol], axis=1)
            else:
                sh = sl
            acc = acc + sh * wt_ref[dh * 3 + dw][None, None, :]
    out_ref[0] = jnp.tanh(acc)


def _conv_tanh(f_nhwc, wt9):
    nblk = H // _CONV_T
    blk = (1, _CONV_T, W, C)
    return pl.pallas_call(
        _conv_tanh_body,
        grid=(N_CELLS, nblk),
        in_specs=[
            pl.BlockSpec((9, C), lambda n, i: (0, 0)),
            pl.BlockSpec(blk, lambda n, i: (n, jnp.maximum(i - 1, 0), 0, 0)),
            pl.BlockSpec(blk, lambda n, i: (n, i, 0, 0)),
            pl.BlockSpec(blk, lambda n, i: (n, jnp.minimum(i + 1, nblk - 1), 0, 0)),
        ],
        out_specs=pl.BlockSpec(blk, lambda n, i: (n, i, 0, 0)),
        out_shape=jax.ShapeDtypeStruct((N_CELLS, H, W, C), jnp.float32),
    )(wt9, f_nhwc, f_nhwc, f_nhwc)


def _idxwts_body(x_ref, y_ref, idx_ref, wts_ref):
    xv = x_ref[...]
    yv = y_ref[...]
    ix0 = xv * jnp.float32(W - 2) * 0.5
    iy0 = yv * jnp.float32(H - 2)
    for n in range(N_CELLS):
        off = jnp.float32(n / N_CELLS)
        ix = ix0 + off
        iy = iy0 + off
        ixl = jnp.floor(ix)
        iyt = jnp.floor(iy)
        fx = ix - ixl
        fy = iy - iyt
        wxr = 0.5 - 0.5 * jnp.cos(jnp.pi * fx)
        wxl = 1.0 - wxr
        wyb = 0.5 - 0.5 * jnp.cos(jnp.pi * fy)
        wyt = 1.0 - wyb
        ixl_i = jnp.clip(ixl, 0, W - 1).astype(jnp.int32)
        ixr_i = jnp.clip(ixl + 1.0, 0, W - 1).astype(jnp.int32)
        iyt_i = jnp.clip(iyt, 0, H - 1).astype(jnp.int32)
        iyb_i = jnp.clip(iyt + 1.0, 0, H - 1).astype(jnp.int32)
        base = n * H * W
        idx_ref[4 * n + 0] = base + iyt_i * W + ixl_i
        idx_ref[4 * n + 1] = base + iyt_i * W + ixr_i
        idx_ref[4 * n + 2] = base + iyb_i * W + ixl_i
        idx_ref[4 * n + 3] = base + iyb_i * W + ixr_i
        wts_ref[4 * n + 0] = wxl * wyt
        wts_ref[4 * n + 1] = wxr * wyt
        wts_ref[4 * n + 2] = wxl * wyb
        wts_ref[4 * n + 3] = wxr * wyb


def _idxwts(xb, yb):
    rows = NPTS // 128
    rblk = 128
    nblk = rows // rblk
    return pl.pallas_call(
        _idxwts_body,
        grid=(nblk,),
        in_specs=[
            pl.BlockSpec((rblk, 128), lambda i: (i, 0)),
            pl.BlockSpec((rblk, 128), lambda i: (i, 0)),
        ],
        out_specs=[
            pl.BlockSpec((8, rblk, 128), lambda i: (0, i, 0)),
            pl.BlockSpec((8, rblk, 128), lambda i: (0, i, 0)),
        ],
        out_shape=[
            jax.ShapeDtypeStruct((8, rows, 128), jnp.int32),
            jax.ShapeDtypeStruct((8, rows, 128), jnp.float32),
        ],
    )(xb, yb)


def _sc_interp_body(table_hbm, idx_hbm, wts_hbm, out_hbm,
                    idx_v, wts_v, rows_v, out_v,
                    sem_idx, sem_gat, sem_out):
    wid = lax.axis_index("s") * NC + lax.axis_index("c")
    iota = lax.iota(jnp.int32, 16)
    n = N_CHUNKS

    def idx_copies(k, b):
        g4 = wid * (N_CHUNKS * GROUPS) + k * GROUPS
        return (
            pltpu.make_async_copy(
                idx_hbm.at[pl.ds(g4, GROUPS)], idx_v.at[b], sem_idx),
            pltpu.make_async_copy(
                wts_hbm.at[pl.ds(g4 * 8, GROUPS * 8)],
                wts_v.at[pl.ds(b * GROUPS * 8, GROUPS * 8)], sem_idx),
        )

    def gather_copies(b):
        return [pltpu.make_async_copy(
                    table_hbm.at[idx_v.at[b, i]],
                    rows_v.at[pl.ds((b * GROUPS + i) * ROWS_PER_GROUP,
                                    ROWS_PER_GROUP)],
                    sem_gat)
                for i in range(GROUPS)]

    def out_copy(k, b):
        return pltpu.make_async_copy(
            out_v.at[pl.ds(b * CHUNK, CHUNK)],
            out_hbm.at[pl.ds(wid * PTS_PER_W + k * CHUNK, CHUNK)],
            sem_out)

    def compute(b):
        rbase = b * GROUPS * ROWS_PER_GROUP
        for pc in range(GROUPS):
            wvecs = [wts_v[(b * GROUPS + pc) * 8 + j] for j in range(8)]
            rowb = rbase + pc * ROWS_PER_GROUP + iota * 8

            @plsc.parallel_loop(0, C, step=1, unroll=4)
            def cbody(c, wvecs=wvecs, rowb=rowb, pc=pc):
                col = jnp.broadcast_to(c, (16,))
                acc = jnp.zeros((16,), jnp.float32)
                for j in range(8):
                    g = plsc.load_gather(rows_v, [rowb + j, col])
                    acc = acc + wvecs[j] * g
                plsc.store_scatter(
                    out_v, [b * CHUNK + pc * 16 + iota, col], acc)

    for cp in idx_copies(0, 0):
        cp.start()
    for cp in idx_copies(0, 0):
        cp.wait()
    for cp in gather_copies(0):
        cp.start()
    for cp in idx_copies(1, 1):
        cp.start()

    def loop_body(k, carry):
        b = jnp.bitwise_and(k, 1)
        nb = 1 - b
        for cp in gather_copies(b):
            cp.wait()

        @pl.when(k + 1 < n)
        def _():
            for cp in idx_copies(k + 1, nb):
                cp.wait()
            for cp in gather_copies(nb):
                cp.start()

        @pl.when(k >= 2)
        def _():
            out_copy(k - 2, b).wait()

        compute(b)

        @pl.when(k + 2 < n)
        def _():
            for cp in idx_copies(k + 2, b):
                cp.start()

        out_copy(k, b).start()
        return carry

    lax.fori_loop(0, n, loop_body, 0)
    out_copy(n - 2, (n - 2) & 1).wait()
    out_copy(n - 1, (n - 1) & 1).wait()


@functools.lru_cache(maxsize=1)
def _sc_interp():
    return pl.kernel(
        _sc_interp_body,
        out_type=jax.ShapeDtypeStruct((NPTS, C), jnp.float32),
        mesh=plsc.VectorSubcoreMesh(
            core_axis_name="c", subcore_axis_name="s",
            num_cores=NC, num_subcores=NS),
        compiler_params=pltpu.CompilerParams(needs_layout_passes=False),
        scratch_types=[
            pltpu.VMEM((2, GROUPS, 128), jnp.int32),
            pltpu.VMEM((2 * GROUPS * 8, 16), jnp.float32),
            pltpu.VMEM((2 * GROUPS * ROWS_PER_GROUP, C), jnp.float32),
            pltpu.VMEM((2 * CHUNK, C), jnp.float32),
            pltpu.SemaphoreType.DMA,
            pltpu.SemaphoreType.DMA,
            pltpu.SemaphoreType.DMA,
        ],
    )


def kernel(x, y, F_active, conv_w):
    f_nhwc = jnp.transpose(F_active, (0, 2, 3, 1))
    wt9 = conv_w.reshape(C, 9).T
    table = _conv_tanh(f_nhwc, wt9).reshape(N_CELLS * H * W, C)

    xb = x.reshape(NPTS // 128, 128)
    yb = y.reshape(NPTS // 128, 128)
    idx_out, wts_out = _idxwts(xb, yb)

    idx_pj = idx_out.reshape(8, NPTS).T
    idxs = idx_pj.reshape(NPTS // 16, 16 * 8)
    wts_pj = wts_out.reshape(8, NPTS).T
    wtsr = (wts_pj.reshape(NPTS // 16, 16, 8)
            .transpose(0, 2, 1)
            .reshape(NPTS // 16 * 8, 16))

    return _sc_interp()(table, idxs, wtsr)


# speedup vs baseline: 10.2082x; 3.8182x over previous
import functools

import jax
import jax.numpy as jnp
from jax import lax
from jax.experimental import pallas as pl
from jax.experimental.pallas import tpu as pltpu
from jax.experimental.pallas import tpu_sc as plsc

N_CELLS = 2
C = 128
H = 256
W = 256
NPTS = 131072

NC = 2
NS = 16
NW = NC * NS
PTS_PER_W = NPTS // NW
CHUNK = 32
N_CHUNKS = PTS_PER_W // CHUNK
GROUPS = CHUNK // 16
ROWS_PER_GROUP = 16 * 8

_CONV_T = 8


def _conv_tanh_body(wt_ref, prev_ref, cur_ref, nxt_ref, out_ref):
    i = pl.program_id(1)
    ni = pl.num_programs(1)
    t = _CONV_T
    prev = prev_ref[0]
    cur = cur_ref[0]
    nxt = nxt_ref[0]
    top = jnp.where(i > 0, prev[t - 1:t], 0.0)
    bot = jnp.where(i < ni - 1, nxt[0:1], 0.0)
    ext = jnp.concatenate([top, cur, bot], axis=0)
    acc = jnp.zeros((t, W, C), jnp.float32)
    zcol = jnp.zeros((t, 1, C), jnp.float32)
    for dh in range(3):
        sl = ext[dh:dh + t]
        for dw in range(3):
            if dw == 0:
                sh = jnp.concatenate([zcol, sl[:, :W - 1, :]], axis=1)
            elif dw == 2:
                sh = jnp.concatenate([sl[:, 1:, :], zcol], axis=1)
            else:
                sh = sl
            acc = acc + sh * wt_ref[dh * 3 + dw][None, None, :]
    out_ref[0] = jnp.tanh(acc)


def _conv_tanh(f_nhwc, wt9):
    nblk = H // _CONV_T
    blk = (1, _CONV_T, W, C)
    return pl.pallas_call(
        _conv_tanh_body,
        grid=(N_CELLS, nblk),
        in_specs=[
            pl.BlockSpec((9, C), lambda n, i: (0, 0)),
            pl.BlockSpec(blk, lambda n, i: (n, jnp.maximum(i - 1, 0), 0, 0)),
            pl.BlockSpec(blk, lambda n, i: (n, i, 0, 0)),
            pl.BlockSpec(blk, lambda n, i: (n, jnp.minimum(i + 1, nblk - 1), 0, 0)),
        ],
        out_specs=pl.BlockSpec(blk, lambda n, i: (n, i, 0, 0)),
        out_shape=jax.ShapeDtypeStruct((N_CELLS, H, W, C), jnp.float32),
    )(wt9, f_nhwc, f_nhwc, f_nhwc)


def _idxwts_body(x_ref, y_ref, idx_ref, wts_ref):
    xv = x_ref[...]
    yv = y_ref[...]
    ix0 = xv * jnp.float32(W - 2) * 0.5
    iy0 = yv * jnp.float32(H - 2)
    for n in range(N_CELLS):
        off = jnp.float32(n / N_CELLS)
        ix = ix0 + off
        iy = iy0 + off
        ixl = jnp.floor(ix)
        iyt = jnp.floor(iy)
        fx = ix - ixl
        fy = iy - iyt
        wxr = 0.5 - 0.5 * jnp.cos(jnp.pi * fx)
        wxl = 1.0 - wxr
        wyb = 0.5 - 0.5 * jnp.cos(jnp.pi * fy)
        wyt = 1.0 - wyb
        ixl_i = jnp.clip(ixl, 0, W - 1).astype(jnp.int32)
        ixr_i = jnp.clip(ixl + 1.0, 0, W - 1).astype(jnp.int32)
        iyt_i = jnp.clip(iyt, 0, H - 1).astype(jnp.int32)
        iyb_i = jnp.clip(iyt + 1.0, 0, H - 1).astype(jnp.int32)
        base = n * H * W
        idx_ref[4 * n + 0] = base + iyt_i * W + ixl_i
        idx_ref[4 * n + 1] = base + iyt_i * W + ixr_i
        idx_ref[4 * n + 2] = base + iyb_i * W + ixl_i
        idx_ref[4 * n + 3] = base + iyb_i * W + ixr_i
        wts_ref[4 * n + 0] = wxl * wyt
        wts_ref[4 * n + 1] = wxr * wyt
        wts_ref[4 * n + 2] = wxl * wyb
        wts_ref[4 * n + 3] = wxr * wyb


def _idxwts(xb, yb):
    rows = NPTS // 128
    rblk = 128
    nblk = rows // rblk
    return pl.pallas_call(
        _idxwts_body,
        grid=(nblk,),
        in_specs=[
            pl.BlockSpec((rblk, 128), lambda i: (i, 0)),
            pl.BlockSpec((rblk, 128), lambda i: (i, 0)),
        ],
        out_specs=[
            pl.BlockSpec((8, rblk, 128), lambda i: (0, i, 0)),
            pl.BlockSpec((8, rblk, 128), lambda i: (0, i, 0)),
        ],
        out_shape=[
            jax.ShapeDtypeStruct((8, rows, 128), jnp.int32),
            jax.ShapeDtypeStruct((8, rows, 128), jnp.float32),
        ],
    )(xb, yb)


def _sc_interp_body(table_hbm, idx_hbm, wts_hbm, out_hbm,
                    idx_v, wts_v, rows_v, out_v,
                    sem_idx, sem_gat, sem_out):
    wid = lax.axis_index("s") * NC + lax.axis_index("c")
    iota = lax.iota(jnp.int32, 16)
    n = N_CHUNKS

    def idx_copies(k, b):
        g4 = wid * (N_CHUNKS * GROUPS) + k * GROUPS
        return (
            pltpu.make_async_copy(
                idx_hbm.at[pl.ds(g4, GROUPS)], idx_v.at[b], sem_idx),
            pltpu.make_async_copy(
                wts_hbm.at[pl.ds(wid * PTS_PER_W + k * CHUNK, CHUNK)],
                wts_v.at[pl.ds(b * CHUNK, CHUNK)], sem_idx),
        )

    def gather_copies(b):
        return [pltpu.make_async_copy(
                    table_hbm.at[idx_v.at[b, i]],
                    rows_v.at[pl.ds((b * GROUPS + i) * ROWS_PER_GROUP,
                                    ROWS_PER_GROUP)],
                    sem_gat)
                for i in range(GROUPS)]

    def out_copy(k, b):
        return pltpu.make_async_copy(
            out_v.at[pl.ds(b * CHUNK, CHUNK)],
            out_hbm.at[pl.ds(wid * PTS_PER_W + k * CHUNK, CHUNK)],
            sem_out)

    def compute(b):
        boff = b * CHUNK
        rbase = b * GROUPS * ROWS_PER_GROUP

        @plsc.parallel_loop(0, CHUNK, step=1, unroll=2)
        def pbody(p):
            r0 = rbase + p * 8
            wrow = boff + p
            ws = [wts_v[wrow, pl.ds(j * 16, 16)] for j in range(8)]
            for v in range(8):
                sl = pl.ds(v * 16, 16)
                acc = ws[0] * rows_v[r0, sl]
                for j in range(1, 8):
                    acc = acc + ws[j] * rows_v[r0 + j, sl]
                out_v[wrow, sl] = acc

    for cp in idx_copies(0, 0):
        cp.start()
    for cp in idx_copies(0, 0):
        cp.wait()
    for cp in gather_copies(0):
        cp.start()
    for cp in idx_copies(1, 1):
        cp.start()

    def loop_body(k, carry):
        b = jnp.bitwise_and(k, 1)
        nb = 1 - b
        for cp in gather_copies(b):
            cp.wait()

        @pl.when(k + 1 < n)
        def _():
            for cp in idx_copies(k + 1, nb):
                cp.wait()
            for cp in gather_copies(nb):
                cp.start()

        @pl.when(k >= 2)
        def _():
            out_copy(k - 2, b).wait()

        compute(b)

        @pl.when(k + 2 < n)
        def _():
            for cp in idx_copies(k + 2, b):
                cp.start()

        out_copy(k, b).start()
        return carry

    lax.fori_loop(0, n, loop_body, 0)
    out_copy(n - 2, (n - 2) & 1).wait()
    out_copy(n - 1, (n - 1) & 1).wait()


@functools.lru_cache(maxsize=1)
def _sc_interp():
    return pl.kernel(
        _sc_interp_body,
        out_type=jax.ShapeDtypeStruct((NPTS, C), jnp.float32),
        mesh=plsc.VectorSubcoreMesh(
            core_axis_name="c", subcore_axis_name="s",
            num_cores=NC, num_subcores=NS),
        compiler_params=pltpu.CompilerParams(needs_layout_passes=False),
        scratch_types=[
            pltpu.VMEM((2, GROUPS, 128), jnp.int32),
            pltpu.VMEM((2 * CHUNK, C), jnp.float32),
            pltpu.VMEM((2 * GROUPS * ROWS_PER_GROUP, C), jnp.float32),
            pltpu.VMEM((2 * CHUNK, C), jnp.float32),
            pltpu.SemaphoreType.DMA,
            pltpu.SemaphoreType.DMA,
            pltpu.SemaphoreType.DMA,
        ],
    )


def kernel(x, y, F_active, conv_w):
    f_nhwc = jnp.transpose(F_active, (0, 2, 3, 1))
    wt9 = conv_w.reshape(C, 9).T
    table = _conv_tanh(f_nhwc, wt9).reshape(N_CELLS * H * W, C)

    xb = x.reshape(NPTS // 128, 128)
    yb = y.reshape(NPTS // 128, 128)
    idx_out, wts_out = _idxwts(xb, yb)

    idx_pj = idx_out.reshape(8, NPTS).T
    idxs = idx_pj.reshape(NPTS // 16, 16 * 8)
    wtsr = jnp.repeat(wts_out.reshape(8, NPTS).T, 16, axis=1)

    return _sc_interp()(table, idxs, wtsr)
